# baseline (speedup 1.0000x reference)
import jax
import jax.numpy as jnp
from jax import lax
from jax.experimental import pallas as pl
from jax.experimental.pallas import tpu as pltpu
from jax.experimental.pallas import tpu_sc as plsc

B, C, Hp, Wp = 4, 96, 192, 192
H, W = 384, 384

NC, NS = 2, 16
NW = NC * NS

ROWS_PER_CHUNK = 32
IN_CH = ROWS_PER_CHUNK * Wp
OUT_ROWS = 2 * ROWS_PER_CHUNK
OUT_CH = OUT_ROWS * W
N_CHUNKS = (B * C * Hp) // ROWS_PER_CHUNK
CHUNKS_PER_TILE = N_CHUNKS // NW
SUBS_PER_PLANE = Hp // ROWS_PER_CHUNK
PLANES_PER_TILE = CHUNKS_PER_TILE // SUBS_PER_PLANE
UNROLL = 2
VR = Wp // 16


def _unpool_body(x_hbm, idx_hbm, out_hbm,
                 xv0, xv1, xv2, iv0, iv1, iv2, buf0, buf1,
                 isem0, isem1, isem2, osem0, osem1):
    xvs = (xv0, xv1, xv2)
    ivs = (iv0, iv1, iv2)
    bufs = (buf0, buf1)
    isems = (isem0, isem1, isem2)
    osems = (osem0, osem1)

    wid = lax.axis_index("c") * NS + lax.axis_index("s")
    zero16 = jnp.zeros((16,), jnp.float32)

    def bc(plane):
        return lax.div(plane, C), lax.rem(plane, C)

    def in_src(hbm, plane, sub):
        b, ch = bc(plane)
        return hbm.at[b, ch, pl.ds(sub * ROWS_PER_CHUNK, ROWS_PER_CHUNK), :]

    def issue_in(plane, sub, s):
        pltpu.async_copy(in_src(x_hbm, plane, sub), xvs[s], isems[s])
        pltpu.async_copy(in_src(idx_hbm, plane, sub), ivs[s], isems[s])

    def wait_in(plane, sub, s):
        pltpu.make_async_copy(in_src(x_hbm, plane, sub), xvs[s],
                              isems[s]).wait()
        pltpu.make_async_copy(in_src(idx_hbm, plane, sub), ivs[s],
                              isems[s]).wait()

    def out_dst(plane, sub):
        b, ch = bc(plane)
        return out_hbm.at[b, ch, pl.ds(sub * OUT_ROWS, OUT_ROWS), :]

    def scatter_chunk(iv, buf, vals_of):
        @plsc.parallel_loop(0, ROWS_PER_CHUNK, 1, unroll=UNROLL)
        def _(r):
            for jj in range(VR):
                pvec = iv[r, pl.ds(jj * 16, 16)]
                rvec = lax.shift_right_logical(pvec, 16)
                cvec = lax.bitwise_and(pvec, 0xFFFF)
                plsc.store_scatter(buf, [rvec, cvec], vals_of(r, jj))

    plane0 = wid * PLANES_PER_TILE
    for j in range(3):
        issue_in(plane0, j, j)

    def outer(u, _):
        plane = plane0 + u
        for t6 in range(SUBS_PER_PLANE):
            t = u * SUBS_PER_PLANE + t6
            bs = t6 % 2
            ins = t6 % 3
            sub_prev = (t6 - 2) % SUBS_PER_PLANE
            plane_prev = plane - 1 if t6 < 2 else plane

            @pl.when(t >= 2)
            def _():
                pltpu.make_async_copy(
                    bufs[bs], out_dst(plane_prev, sub_prev), osems[bs]).wait()
                ivp = ivs[(t6 - 2) % 3]
                scatter_chunk(ivp, bufs[bs], lambda r, jj: zero16)

            @pl.when(t < 2)
            def _():
                @plsc.parallel_loop(0, OUT_ROWS, 1, unroll=UNROLL)
                def _(r):
                    for jj in range(W // 16):
                        bufs[bs][r, pl.ds(jj * 16, 16)] = zero16

            @pl.when(jnp.logical_and(t >= 2, t < CHUNKS_PER_TILE - 1))
            def _():
                sub_next = (t6 + 1) % SUBS_PER_PLANE
                plane_next = plane + 1 if t6 == SUBS_PER_PLANE - 1 else plane
                issue_in(plane_next, sub_next, (t6 + 1) % 3)

            wait_in(plane, t6, ins)
            scatter_chunk(ivs[ins], bufs[bs],
                          lambda r, jj: xvs[ins][r, pl.ds(jj * 16, 16)])

            pltpu.async_copy(bufs[bs], out_dst(plane, t6), osems[bs])
        return ()

    lax.fori_loop(0, PLANES_PER_TILE, outer, ())

    last_plane = plane0 + PLANES_PER_TILE - 1
    for t6 in (SUBS_PER_PLANE - 2, SUBS_PER_PLANE - 1):
        pltpu.make_async_copy(
            bufs[t6 % 2], out_dst(last_plane, t6), osems[t6 % 2]).wait()


@jax.jit
def kernel(x, indices):
    row = indices // W
    col = indices - row * W
    packed = lax.shift_left(lax.bitwise_and(row, 63), 16) | col
    mesh = plsc.VectorSubcoreMesh(core_axis_name="c", subcore_axis_name="s",
                                  num_cores=NC, num_subcores=NS)
    run = pl.kernel(
        _unpool_body,
        out_type=jax.ShapeDtypeStruct((B, C, H, W), jnp.float32),
        mesh=mesh,
        scratch_types=[
            pltpu.VMEM((ROWS_PER_CHUNK, Wp), jnp.float32),
            pltpu.VMEM((ROWS_PER_CHUNK, Wp), jnp.float32),
            pltpu.VMEM((ROWS_PER_CHUNK, Wp), jnp.float32),
            pltpu.VMEM((ROWS_PER_CHUNK, Wp), jnp.int32),
            pltpu.VMEM((ROWS_PER_CHUNK, Wp), jnp.int32),
            pltpu.VMEM((ROWS_PER_CHUNK, Wp), jnp.int32),
            pltpu.VMEM((OUT_ROWS, W), jnp.float32),
            pltpu.VMEM((OUT_ROWS, W), jnp.float32),
            pltpu.SemaphoreType.DMA,
            pltpu.SemaphoreType.DMA,
            pltpu.SemaphoreType.DMA,
            pltpu.SemaphoreType.DMA,
            pltpu.SemaphoreType.DMA,
        ],
        compiler_params=pltpu.CompilerParams(needs_layout_passes=False),
    )
    return run(x, packed)
